# baseline (speedup 1.0000x reference)
import jax
import jax.numpy as jnp
from jax.experimental import pallas as pl
from jax.experimental.pallas import tpu as pltpu


def _make_body(T, S, A, H, H2):
    f32 = jnp.float32

    pieces = _split_pieces(S)
    P = len(pieces)

    def body(*refs):
        sr_refs = refs[:P]
        (hds_ref, acts_ref, vals_ref, w1at_ref, w1ct_ref, b1a_ref, b1c_ref,
         w2a_ref, w2ct_ref, b2c_ref) = refs[P:P + 10]
        out_ref, val_ref, w4s_ref, wfs_ref = refs[P + 10:]
        bb = sr_refs[0].shape[-1]

        @pl.when(pl.program_id(0) == 0)
        def _build_interleaved_weights():
            w1fT = jnp.concatenate(
                [w1at_ref[...], w1ct_ref[...]], axis=0)
            wmain = w1fT[:, :S]
            b1row = jnp.concatenate(
                [b1a_ref[...], b1c_ref[...]], axis=1)
            ei = jax.lax.broadcasted_iota(jnp.int32, (H2, H2), 0)
            ej = jax.lax.broadcasted_iota(jnp.int32, (H2, H2), 1)
            eye2h = (ei == ej).astype(f32)
            b1col = jax.lax.dot_general(
                eye2h, b1row, (((1,), (1,)), ((), ())),
                preferred_element_type=f32)
            wsmall = jnp.concatenate(
                [w1fT[:, S:S + 3], b1col], axis=1)
            for t in range(T):
                ci = jax.lax.broadcasted_iota(jnp.int32, (S, S * T), 1)
                ri = jax.lax.broadcasted_iota(jnp.int32, (S, S * T), 0)
                sel = (ci == T * ri + t).astype(f32)
                w4s_ref[H2 * t:H2 * (t + 1), :] = jnp.dot(
                    wmain, sel, preferred_element_type=f32)
                cif = jax.lax.broadcasted_iota(jnp.int32, (4, 4 * T), 1)
                rif = jax.lax.broadcasted_iota(jnp.int32, (4, 4 * T), 0)
                self_f = (cif == T * rif + t).astype(f32)
                wfs_ref[H2 * t:H2 * (t + 1), :] = jnp.dot(
                    wsmall, self_f, preferred_element_type=f32)

        hp4 = None
        c0 = 0
        for p, (sz, _idx) in enumerate(pieces):
            term = jnp.dot(w4s_ref[:, c0:c0 + sz * T],
                           sr_refs[p][...].reshape(sz * T, bb),
                           preferred_element_type=f32)
            hp4 = term if hp4 is None else hp4 + term
            c0 += sz * T

        feats = jnp.concatenate(
            [hds_ref[...], acts_ref[...], vals_ref[...],
             jnp.ones((T, bb), f32)], axis=0)
        hp4 = hp4 + jnp.dot(wfs_ref[...], feats, preferred_element_type=f32)

        w2a = w2a_ref[...]

        outs = []
        for t in range(T):
            ha = jnp.tanh(hp4[H2 * t:H2 * t + H, :])
            outs.append(jax.lax.dot_general(
                w2a, ha, (((0,), (0,)), ((), ())),
                preferred_element_type=f32))

        hc = jnp.tanh(hp4[H:H2, :])
        val_ref[...] = (jnp.dot(w2ct_ref[...], hc, preferred_element_type=f32)
                        + b2c_ref[...])

        m = outs[0]
        for t in range(1, T):
            m = jnp.maximum(m, outs[t])
        se = jnp.exp(outs[0] - m)
        for t in range(1, T):
            se = se + jnp.exp(outs[t] - m)
        lse = m + jnp.log(se)
        for t in range(T):
            out_ref[t, :, :] = outs[t] - lse

    return body


def _split_pieces(S):
    for rest in (S // 5, S // 4, S // 3, S // 2):
        big = S - rest
        if rest > 0 and big > 0 and big % rest == 0:
            return [(big, 0), (rest, big // rest)]
    return [(S, 0)]


def _pick_b_block(B):
    for cand in (4096, 2048, 1024, 512, 256, 128):
        if B % cand == 0 and (B // cand) >= 2:
            return cand
    return B


def kernel(w1a, b1a, w2a, b2a, w1c, b1c, w2c, b2c, SR, HDs, acts, values):
    f32 = jnp.float32
    B, T, S = SR.shape
    H = w1a.shape[1]
    H2 = 2 * H
    A = w2a.shape[1]

    w1at = w1a.astype(f32).T
    w1ct = w1c.astype(f32).T
    w2ct = w2c.astype(f32).T
    b2ct = b2c.astype(f32)

    SRt = jnp.transpose(SR.astype(f32), (2, 1, 0))
    HDst = HDs.astype(f32).T
    actst = acts.astype(f32).T
    valst = values.astype(f32).T

    pieces = _split_pieces(S)
    P = len(pieces)
    bb = _pick_b_block(B)
    nb = B // bb
    lane_tile2 = lambda i: (0, i)
    full2 = lambda i: (0, 0)

    out_tab, val = pl.pallas_call(
        _make_body(T, S, A, H, H2),
        grid=(nb,),
        in_specs=[
            pl.BlockSpec((sz, T, bb), lambda i, _idx=idx: (_idx, 0, i))
            for sz, idx in pieces
        ] + [
            pl.BlockSpec((T, bb), lane_tile2),
            pl.BlockSpec((T, bb), lane_tile2),
            pl.BlockSpec((T, bb), lane_tile2),
            pl.BlockSpec((H, S + 3), full2),
            pl.BlockSpec((H, S + 3), full2),
            pl.BlockSpec((1, H), full2),
            pl.BlockSpec((1, H), full2),
            pl.BlockSpec((H, A), full2),
            pl.BlockSpec((1, H), full2),
            pl.BlockSpec((1, 1), full2),
        ],
        out_specs=[
            pl.BlockSpec((T, A, bb), lambda i: (0, 0, i)),
            pl.BlockSpec((1, bb), lane_tile2),
        ],
        out_shape=[
            jax.ShapeDtypeStruct((T, A, B), f32),
            jax.ShapeDtypeStruct((1, B), f32),
        ],
        scratch_shapes=[
            pltpu.VMEM((T * H2, S * T), f32),
            pltpu.VMEM((T * H2, 4 * T), f32),
        ],
        compiler_params=pltpu.CompilerParams(
            dimension_semantics=("arbitrary",)),
    )(*([SRt] * P), HDst, actst, valst, w1at, w1ct, b1a.astype(f32),
      b1c.astype(f32), w2a.astype(f32), w2ct, b2ct)

    logp = jnp.transpose(out_tab, (2, 0, 1))
    value = val.reshape(B)
    return logp, value


# speedup vs baseline: 1.1859x; 1.1859x over previous
import jax
import jax.numpy as jnp
from jax.experimental import pallas as pl
from jax.experimental.pallas import tpu as pltpu


def _make_body(T, S, A, H, H2):
    f32 = jnp.float32

    def body(sr_ref, hds_ref, acts_ref, vals_ref,
             w1at_ref, w1ct_ref, b1a_ref, b1c_ref, w2a_ref, w2ct_ref,
             b2c_ref, out_ref, val_ref, w4s_ref, wfs_ref):
        bb = sr_ref.shape[-1]

        @pl.when(pl.program_id(0) == 0)
        def _build_interleaved_weights():
            w1fT = jnp.concatenate(
                [w1at_ref[...], w1ct_ref[...]], axis=0)
            wmain = w1fT[:, :S]
            b1row = jnp.concatenate(
                [b1a_ref[...], b1c_ref[...]], axis=1)
            ei = jax.lax.broadcasted_iota(jnp.int32, (H2, H2), 0)
            ej = jax.lax.broadcasted_iota(jnp.int32, (H2, H2), 1)
            eye2h = (ei == ej).astype(f32)
            b1col = jax.lax.dot_general(
                eye2h, b1row, (((1,), (1,)), ((), ())),
                preferred_element_type=f32)
            wsmall = jnp.concatenate(
                [w1fT[:, S:S + 3], b1col], axis=1)
            for t in range(T):
                ci = jax.lax.broadcasted_iota(jnp.int32, (S, S * T), 1)
                ri = jax.lax.broadcasted_iota(jnp.int32, (S, S * T), 0)
                sel = (ci == T * ri + t).astype(f32)
                w4s_ref[H2 * t:H2 * (t + 1), :] = jnp.dot(
                    wmain, sel, preferred_element_type=f32)
                cif = jax.lax.broadcasted_iota(jnp.int32, (4, 4 * T), 1)
                rif = jax.lax.broadcasted_iota(jnp.int32, (4, 4 * T), 0)
                self_f = (cif == T * rif + t).astype(f32)
                wfs_ref[H2 * t:H2 * (t + 1), :] = jnp.dot(
                    wsmall, self_f, preferred_element_type=f32)

        x2d = sr_ref[...].reshape(S * T, bb)
        hp4 = jnp.dot(w4s_ref[...], x2d, preferred_element_type=f32)

        feats = jnp.concatenate(
            [hds_ref[...], acts_ref[...], vals_ref[...],
             jnp.ones((T, bb), f32)], axis=0)
        hp4 = hp4 + jnp.dot(wfs_ref[...], feats, preferred_element_type=f32)

        w2a = w2a_ref[...]

        outs = []
        for t in range(T):
            ha = jnp.tanh(hp4[H2 * t:H2 * t + H, :])
            outs.append(jax.lax.dot_general(
                w2a, ha, (((0,), (0,)), ((), ())),
                preferred_element_type=f32))

        hc = jnp.tanh(hp4[H:H2, :])
        val_ref[...] = (jnp.dot(w2ct_ref[...], hc, preferred_element_type=f32)
                        + b2c_ref[...])

        m = outs[0]
        for t in range(1, T):
            m = jnp.maximum(m, outs[t])
        se = jnp.exp(outs[0] - m)
        for t in range(1, T):
            se = se + jnp.exp(outs[t] - m)
        lse = m + jnp.log(se)
        for t in range(T):
            out_ref[t, :, :] = outs[t] - lse

    return body


def _pick_b_block(B):
    for cand in (4096, 2048, 1024, 512, 256, 128):
        if B % cand == 0 and (B // cand) >= 2:
            return cand
    return B


def kernel(w1a, b1a, w2a, b2a, w1c, b1c, w2c, b2c, SR, HDs, acts, values):
    f32 = jnp.float32
    B, T, S = SR.shape
    H = w1a.shape[1]
    H2 = 2 * H
    A = w2a.shape[1]

    w1at = w1a.astype(f32).T
    w1ct = w1c.astype(f32).T
    w2ct = w2c.astype(f32).T
    b2ct = b2c.astype(f32)

    SRt = jnp.transpose(SR.astype(f32), (2, 1, 0))
    HDst = HDs.astype(f32).T
    actst = acts.astype(f32).T
    valst = values.astype(f32).T

    bb = _pick_b_block(B)
    nb = B // bb
    lane_tile2 = lambda i: (0, i)
    full2 = lambda i: (0, 0)

    out_tab, val = pl.pallas_call(
        _make_body(T, S, A, H, H2),
        grid=(nb,),
        in_specs=[
            pl.BlockSpec((S, T, bb), lambda i: (0, 0, i)),
            pl.BlockSpec((T, bb), lane_tile2),
            pl.BlockSpec((T, bb), lane_tile2),
            pl.BlockSpec((T, bb), lane_tile2),
            pl.BlockSpec((H, S + 3), full2),
            pl.BlockSpec((H, S + 3), full2),
            pl.BlockSpec((1, H), full2),
            pl.BlockSpec((1, H), full2),
            pl.BlockSpec((H, A), full2),
            pl.BlockSpec((1, H), full2),
            pl.BlockSpec((1, 1), full2),
        ],
        out_specs=[
            pl.BlockSpec((T, A, bb), lambda i: (0, 0, i)),
            pl.BlockSpec((1, bb), lane_tile2),
        ],
        out_shape=[
            jax.ShapeDtypeStruct((T, A, B), f32),
            jax.ShapeDtypeStruct((1, B), f32),
        ],
        scratch_shapes=[
            pltpu.VMEM((T * H2, S * T), f32),
            pltpu.VMEM((T * H2, 4 * T), f32),
        ],
        compiler_params=pltpu.CompilerParams(
            dimension_semantics=("arbitrary",)),
    )(SRt, HDst, actst, valst, w1at, w1ct, b1a.astype(f32), b1c.astype(f32),
      w2a.astype(f32), w2ct, b2ct)

    logp = jnp.transpose(out_tab, (2, 0, 1))
    value = val.reshape(B)
    return logp, value
